# baseline (speedup 1.0000x reference)
import functools

import jax
import jax.numpy as jnp
from jax import lax
from jax.experimental import pallas as pl
from jax.experimental.pallas import tpu as pltpu
from jax.experimental.pallas import tpu_sc as plsc

_NC = 2
_NS = 16
_NW = _NC * _NS
_GS = 16


def _prep_body(at_ref, tm_ref, idx_ref, table_ref):
    na = at_ref.shape[0]
    best = at_ref[0]
    bidx = jnp.zeros(best.shape, jnp.int32)
    for j in range(1, na):
        v = at_ref[j]
        m = v > best
        best = jnp.where(m, v, best)
        bidx = jnp.where(m, j, bidx)
    idx_ref[...] = bidx
    t = tm_ref[...]
    t = t - jnp.max(t, axis=-1, keepdims=True)
    e = jnp.exp(t)
    table_ref[...] = e / jnp.sum(e, axis=-1, keepdims=True)


def _make_expand(n_rows, n_tab, k, rpw):
    ngroup = rpw // _GS
    mesh = plsc.VectorSubcoreMesh(core_axis_name="c", subcore_axis_name="s")

    @functools.partial(
        pl.kernel,
        out_type=jax.ShapeDtypeStruct((n_rows, k, k), jnp.float32),
        mesh=mesh,
        scratch_types=[
            pltpu.VMEM((rpw,), jnp.int32),
            pltpu.VMEM((n_tab, k, k), jnp.float32),
            pltpu.SemaphoreType.DMA,
        ],
    )
    def expand(table_hbm, idx_hbm, out_hbm, idx_v, table_v, sem):
        wid = lax.axis_index("s") * _NC + lax.axis_index("c")
        base = wid * rpw
        pltpu.sync_copy(idx_hbm.at[pl.ds(base, rpw)], idx_v)
        pltpu.sync_copy(table_hbm, table_v)

        def drain_group(t):
            for i in range(_GS):
                pltpu.make_async_copy(
                    table_v.at[0], out_hbm.at[base + t * _GS + i],
                    sem).wait()

        def body(t, _):
            r0 = t * _GS
            avec = idx_v[pl.ds(r0, _GS)]
            for i in range(_GS):
                a = avec[i]
                pltpu.async_copy(table_v.at[a], out_hbm.at[base + r0 + i],
                                 sem)
            drain_group(t)
            return 0

        lax.fori_loop(0, ngroup, body, 0)

    return expand


def kernel(action, transition_matrix):
    dim = action.ndim
    if dim == 2:
        action = action[None]
    s, b, na = action.shape
    c, _, k, _ = transition_matrix.shape
    n = s * b

    prep = pl.pallas_call(
        _prep_body,
        out_shape=(
            jax.ShapeDtypeStruct((n,), jnp.int32),
            jax.ShapeDtypeStruct((c, na, k, k), jnp.float32),
        ),
    )
    idx, table = prep(action.reshape(n, na).T, transition_matrix)

    rows = idx
    if c > 1:
        rows = (rows[:, None]
                + jnp.arange(c, dtype=jnp.int32)[None, :] * na).reshape(-1)
    nr = n * c

    npad = -(-nr // (_NW * _GS)) * (_NW * _GS)
    if npad != nr:
        rows = jnp.pad(rows, (0, npad - nr))
    rpw = npad // _NW
    table_flat = table.reshape(c * na, k, k)

    out_flat = _make_expand(npad, c * na, k, rpw)(table_flat, rows)
    if npad != nr:
        out_flat = out_flat[:nr]
    return out_flat


# speedup vs baseline: 1.3348x; 1.3348x over previous
import functools

import jax
import jax.numpy as jnp
from jax import lax
from jax.experimental import pallas as pl
from jax.experimental.pallas import tpu as pltpu
from jax.experimental.pallas import tpu_sc as plsc

_NC = 2
_NS = 16
_NW = _NC * _NS
_GS = 16


def _prep_body(at_ref, tm_ref, idx_ref, table_ref):
    na = at_ref.shape[0]
    best = at_ref[0]
    bidx = jnp.zeros(best.shape, jnp.int32)
    for j in range(1, na):
        v = at_ref[j]
        m = v > best
        best = jnp.where(m, v, best)
        bidx = jnp.where(m, j, bidx)
    idx_ref[...] = bidx
    t = tm_ref[...]
    t = t - jnp.max(t, axis=-1, keepdims=True)
    e = jnp.exp(t)
    table_ref[...] = e / jnp.sum(e, axis=-1, keepdims=True)


def _make_expand(n_rows, n_tab, k, rpw):
    ngroup = rpw // _GS
    mesh = plsc.VectorSubcoreMesh(core_axis_name="c", subcore_axis_name="s")

    @functools.partial(
        pl.kernel,
        out_type=jax.ShapeDtypeStruct((n_rows, k, k), jnp.float32),
        mesh=mesh,
        scratch_types=[
            pltpu.VMEM((rpw,), jnp.int32),
            pltpu.VMEM((n_tab, k, k), jnp.float32),
            pltpu.SemaphoreType.DMA,
        ],
        compiler_params=pltpu.CompilerParams(use_tc_tiling_on_sc=True),
    )
    def expand(table_hbm, idx_hbm, out_hbm, idx_v, table_v, sem):
        wid = lax.axis_index("s") * _NC + lax.axis_index("c")
        base = wid * rpw
        pltpu.sync_copy(idx_hbm.at[pl.ds(base, rpw)], idx_v)
        pltpu.sync_copy(table_hbm, table_v)

        def drain_group(t):
            for i in range(_GS):
                pltpu.make_async_copy(
                    table_v.at[0], out_hbm.at[base + t * _GS + i],
                    sem).wait()

        def body(t, _):
            r0 = t * _GS
            avec = idx_v[pl.ds(r0, _GS)]
            for i in range(_GS):
                a = avec[i]
                pltpu.async_copy(table_v.at[a], out_hbm.at[base + r0 + i],
                                 sem)
            drain_group(t)
            return 0

        lax.fori_loop(0, ngroup, body, 0)

    return expand


def kernel(action, transition_matrix):
    dim = action.ndim
    if dim == 2:
        action = action[None]
    s, b, na = action.shape
    c, _, k, _ = transition_matrix.shape
    n = s * b

    prep = pl.pallas_call(
        _prep_body,
        out_shape=(
            jax.ShapeDtypeStruct((n,), jnp.int32),
            jax.ShapeDtypeStruct((c, na, k, k), jnp.float32),
        ),
    )
    idx, table = prep(action.reshape(n, na).T, transition_matrix)

    rows = idx
    if c > 1:
        rows = (rows[:, None]
                + jnp.arange(c, dtype=jnp.int32)[None, :] * na).reshape(-1)
    nr = n * c

    npad = -(-nr // (_NW * _GS)) * (_NW * _GS)
    if npad != nr:
        rows = jnp.pad(rows, (0, npad - nr))
    rpw = npad // _NW
    table_flat = table.reshape(c * na, k, k)

    out_flat = _make_expand(npad, c * na, k, rpw)(table_flat, rows)
    if npad != nr:
        out_flat = out_flat[:nr]
    prob = out_flat.reshape(s, b, c, k, k)
    if dim == 2:
        prob = prob[0]
    return prob
